# baseline (speedup 1.0000x reference)
import jax
import jax.numpy as jnp
from jax.experimental import pallas as pl
from jax.experimental.pallas import tpu as pltpu

_B = 256
_ND = 64
_NC = 8
_N = _ND + _NC
_G = 64
_HEADS = 4
_DH = 32


def _body(cs_ref, ds_ref, Wc_ref, bc_ref, Wg_ref, bg_ref, wgt_ref,
          W1_ref, b1_ref, Wml_ref, bml_ref,
          Wqkv_ref, Wo_ref, Wp1_ref, bp1_ref, Wp2_ref, bp2_ref,
          mu_ref, lv_ref, pred_ref):
    G = _G
    relu = lambda x: jnp.maximum(x, 0.0)
    bf16 = jnp.bfloat16
    f32 = jnp.float32

    def bdot(a, b, dn=None):
        a16, b16 = a.astype(bf16), b.astype(bf16)
        if dn is None:
            dn = (((a.ndim - 1,), (0,)), ((), ()))
        return jax.lax.dot_general(a16, b16, dn, preferred_element_type=f32)

    def fdot(a, b, dn=None):
        if dn is None:
            dn = (((a.ndim - 1,), (0,)), ((), ()))
        return jax.lax.dot_general(a, b, dn, preferred_element_type=f32)

    cell_list = []
    for c in range(_NC):
        cell_list.append(relu(bdot(cs_ref[c], Wc_ref[c]) + bc_ref[c:c + 1, :]))
    cell3 = jnp.concatenate([x[:, None, :] for x in cell_list], axis=1)

    ds = ds_ref[...]
    sub = relu(bdot(ds.reshape(G * _ND, 128), Wg_ref[...]) + bg_ref[...])
    drug3 = sub.reshape(G, _ND, 128)

    T3 = jax.lax.dot_general(cell3.astype(bf16), wgt_ref[...].astype(bf16),
                             (((2,), (1,)), ((), ())),
                             preferred_element_type=f32)
    s3 = jax.lax.dot_general(T3.astype(bf16), drug3.astype(bf16),
                             (((2,), (2,)), ((0,), (0,))),
                             preferred_element_type=f32)
    gw = jax.nn.sigmoid(s3)
    wmT = jnp.where(gw >= 0.5, gw, 0.0)
    sig = wmT

    dinv_d = jax.lax.rsqrt(1.0 + jnp.sum(sig, axis=1))[:, :, None]
    dinv_c = jax.lax.rsqrt(1.0 + jnp.sum(sig, axis=2))[:, :, None]

    def propagate(Xd3, Xc3):
        Yd = dinv_d * Xd3
        Yc = dinv_c * Xc3
        AXd = dinv_d * (Yd + fdot(wmT, Yc, (((1,), (1,)), ((0,), (0,)))))
        AXc = dinv_c * (Yc + fdot(wmT, Yd, (((2,), (1,)), ((0,), (0,)))))
        return jnp.concatenate([AXd, AXc], axis=1)

    AX = propagate(drug3, cell3).reshape(G * _N, 128)
    H = relu(fdot(AX, W1_ref[...]) + b1_ref[...])
    H3 = H.reshape(G, _N, 256)

    A2 = propagate(H3[:, :_ND, :], H3[:, _ND:, :]).reshape(G * _N, 256)
    mulv = fdot(A2, Wml_ref[...]) + bml_ref[...]
    mu = mulv[:, :128]
    mu_ref[...] = mu.reshape(G, _N, 128)
    lv_ref[...] = mulv[:, 128:].reshape(G, _N, 128)

    qkv = fdot(mu, Wqkv_ref[...]).reshape(G, _N, 3 * 128)
    parts = []
    for h in range(_HEADS):
        q3 = qkv[:, :, h * _DH:(h + 1) * _DH]
        k3 = qkv[:, :, 128 + h * _DH:128 + (h + 1) * _DH]
        v3 = qkv[:, :, 256 + h * _DH:256 + (h + 1) * _DH]
        S = fdot(q3, k3, (((2,), (2,)), ((0,), (0,))))
        S = S - jnp.max(S, axis=-1, keepdims=True)
        E = jnp.exp(S)
        rinv = (1.0 / _N) / jnp.sum(E, axis=-1)
        abar = fdot(rinv, E, (((1,), (1,)), ((0,), (0,))))
        parts.append(fdot(abar, v3, (((1,), (1,)), ((0,), (0,)))))
    obar = jnp.concatenate(parts, axis=-1)

    hg = fdot(obar, Wo_ref[...])
    p1 = relu(fdot(hg, Wp1_ref[...]) + bp1_ref[...])
    pred_ref[...] = jax.nn.sigmoid(fdot(p1, Wp2_ref[...]) + bp2_ref[...])


def kernel(cell_subs, drug_subs, batch, drug_cell_batch, W_cell, b_cell,
           W_gnn, b_gnn, weight, W1, b1, W_mu, b_mu, W_lv, b_lv,
           Wq, Wk, Wv, Wo, Wp1, bp1, Wp2, bp2):
    del batch, drug_cell_batch
    ds3 = drug_subs.reshape(_B, _ND, 128)
    grid = (_B // _G,)

    def full(a):
        return pl.BlockSpec(a.shape, lambda i: (0,) * a.ndim)

    in_specs = [
        pl.BlockSpec((_NC, _G, 128), lambda i: (0, i, 0)),
        pl.BlockSpec((_G, _ND, 128), lambda i: (i, 0, 0)),
        full(W_cell), full(b_cell),
        full(W_gnn), pl.BlockSpec((1, 128), lambda i: (0, 0)),
        full(weight),
        full(W1), pl.BlockSpec((1, 256), lambda i: (0, 0)),
        pl.BlockSpec((256, 256), lambda i: (0, 0)),
        pl.BlockSpec((1, 256), lambda i: (0, 0)),
        pl.BlockSpec((128, 384), lambda i: (0, 0)),
        full(Wo),
        full(Wp1), pl.BlockSpec((1, 256), lambda i: (0, 0)),
        full(Wp2), pl.BlockSpec((1, 1), lambda i: (0, 0)),
    ]
    out_specs = [
        pl.BlockSpec((_G, _N, 128), lambda i: (i, 0, 0)),
        pl.BlockSpec((_G, _N, 128), lambda i: (i, 0, 0)),
        pl.BlockSpec((_G, 1), lambda i: (i, 0)),
    ]
    out_shapes = [
        jax.ShapeDtypeStruct((_B, _N, 128), jnp.float32),
        jax.ShapeDtypeStruct((_B, _N, 128), jnp.float32),
        jax.ShapeDtypeStruct((_B, 1), jnp.float32),
    ]
    mu3, lv3, pred = pl.pallas_call(
        _body,
        grid=grid,
        in_specs=in_specs,
        out_specs=out_specs,
        out_shape=out_shapes,
        compiler_params=pltpu.CompilerParams(
            dimension_semantics=("parallel",)),
    )(cell_subs, ds3, W_cell, b_cell, W_gnn, b_gnn.reshape(1, 128), weight,
      W1, b1.reshape(1, 256),
      jnp.concatenate([W_mu, W_lv], axis=1),
      jnp.concatenate([b_mu, b_lv]).reshape(1, 256),
      jnp.concatenate([Wq / jnp.sqrt(jnp.float32(_DH)), Wk, Wv], axis=1), Wo,
      Wp1, bp1.reshape(1, 256), Wp2, bp2.reshape(1, 1))

    mu_flat = mu3.reshape(-1, 128)
    return (pred, mu_flat, lv3.reshape(-1, 128), mu_flat)


# speedup vs baseline: 1.0790x; 1.0790x over previous
import jax
import jax.numpy as jnp
from jax.experimental import pallas as pl
from jax.experimental.pallas import tpu as pltpu

_B = 256
_ND = 64
_NC = 8
_N = _ND + _NC
_G = 64
_HEADS = 4
_DH = 32


def _body(cs_ref, ds_ref, Wc_ref, bc_ref, Wg_ref, bg_ref, wgt_ref,
          W1_ref, b1_ref, Wml_ref, bml_ref,
          Wqkv_ref, Wo_ref, Wp1_ref, bp1_ref, Wp2_ref, bp2_ref,
          mu_ref, lv_ref, z_ref, pred_ref):
    G = _G
    relu = lambda x: jnp.maximum(x, 0.0)
    bf16 = jnp.bfloat16
    f32 = jnp.float32

    def bdot(a, b, dn=None):
        a16, b16 = a.astype(bf16), b.astype(bf16)
        if dn is None:
            dn = (((a.ndim - 1,), (0,)), ((), ()))
        return jax.lax.dot_general(a16, b16, dn, preferred_element_type=f32)

    def fdot(a, b, dn=None):
        if dn is None:
            dn = (((a.ndim - 1,), (0,)), ((), ()))
        return jax.lax.dot_general(a, b, dn, preferred_element_type=f32)

    cell_list = []
    for c in range(_NC):
        cell_list.append(relu(bdot(cs_ref[c], Wc_ref[c]) + bc_ref[c:c + 1, :]))
    cell3 = jnp.concatenate([x[:, None, :] for x in cell_list], axis=1)

    ds = ds_ref[...]
    sub = relu(bdot(ds.reshape(G * _ND, 128), Wg_ref[...]) + bg_ref[...])
    drug3 = sub.reshape(G, _ND, 128)

    T3 = jax.lax.dot_general(cell3.astype(bf16), wgt_ref[...].astype(bf16),
                             (((2,), (1,)), ((), ())),
                             preferred_element_type=f32)
    s3 = jax.lax.dot_general(T3.astype(bf16), drug3.astype(bf16),
                             (((2,), (2,)), ((0,), (0,))),
                             preferred_element_type=f32)
    gw = jax.nn.sigmoid(s3)
    wmT = jnp.where(gw >= 0.5, gw, 0.0)
    sig = wmT

    dinv_d = jax.lax.rsqrt(1.0 + jnp.sum(sig, axis=1))[:, :, None]
    dinv_c = jax.lax.rsqrt(1.0 + jnp.sum(sig, axis=2))[:, :, None]

    def propagate(Xd3, Xc3):
        Yd = dinv_d * Xd3
        Yc = dinv_c * Xc3
        AXd = dinv_d * (Yd + fdot(wmT, Yc, (((1,), (1,)), ((0,), (0,)))))
        AXc = dinv_c * (Yc + fdot(wmT, Yd, (((2,), (1,)), ((0,), (0,)))))
        return jnp.concatenate([AXd, AXc], axis=1)

    AX = propagate(drug3, cell3).reshape(G * _N, 128)
    H = relu(fdot(AX, W1_ref[...]) + b1_ref[...])
    H3 = H.reshape(G, _N, 256)

    A2 = propagate(H3[:, :_ND, :], H3[:, _ND:, :]).reshape(G * _N, 256)
    mulv = fdot(A2, Wml_ref[...]) + bml_ref[...]
    mu = mulv[:, :128]
    mu_ref[...] = mu.reshape(G, _N, 128)
    z_ref[...] = mu.reshape(G, _N, 128)
    lv_ref[...] = mulv[:, 128:].reshape(G, _N, 128)

    qkv = fdot(mu, Wqkv_ref[...]).reshape(G, _N, 3 * 128)
    parts = []
    for h in range(_HEADS):
        q3 = qkv[:, :, h * _DH:(h + 1) * _DH]
        k3 = qkv[:, :, 128 + h * _DH:128 + (h + 1) * _DH]
        v3 = qkv[:, :, 256 + h * _DH:256 + (h + 1) * _DH]
        S = fdot(q3, k3, (((2,), (2,)), ((0,), (0,))))
        S = S - jnp.max(S, axis=-1, keepdims=True)
        E = jnp.exp(S)
        rinv = (1.0 / _N) / jnp.sum(E, axis=-1)
        abar = fdot(rinv, E, (((1,), (1,)), ((0,), (0,))))
        parts.append(fdot(abar, v3, (((1,), (1,)), ((0,), (0,)))))
    obar = jnp.concatenate(parts, axis=-1)

    hg = fdot(obar, Wo_ref[...])
    p1 = relu(fdot(hg, Wp1_ref[...]) + bp1_ref[...])
    pred_ref[...] = jax.nn.sigmoid(fdot(p1, Wp2_ref[...]) + bp2_ref[...])


def kernel(cell_subs, drug_subs, batch, drug_cell_batch, W_cell, b_cell,
           W_gnn, b_gnn, weight, W1, b1, W_mu, b_mu, W_lv, b_lv,
           Wq, Wk, Wv, Wo, Wp1, bp1, Wp2, bp2):
    del batch, drug_cell_batch
    ds3 = drug_subs.reshape(_B, _ND, 128)
    grid = (_B // _G,)

    def full(a):
        return pl.BlockSpec(a.shape, lambda i: (0,) * a.ndim)

    in_specs = [
        pl.BlockSpec((_NC, _G, 128), lambda i: (0, i, 0)),
        pl.BlockSpec((_G, _ND, 128), lambda i: (i, 0, 0)),
        full(W_cell), full(b_cell),
        full(W_gnn), pl.BlockSpec((1, 128), lambda i: (0, 0)),
        full(weight),
        full(W1), pl.BlockSpec((1, 256), lambda i: (0, 0)),
        pl.BlockSpec((256, 256), lambda i: (0, 0)),
        pl.BlockSpec((1, 256), lambda i: (0, 0)),
        pl.BlockSpec((128, 384), lambda i: (0, 0)),
        full(Wo),
        full(Wp1), pl.BlockSpec((1, 256), lambda i: (0, 0)),
        full(Wp2), pl.BlockSpec((1, 1), lambda i: (0, 0)),
    ]
    out_specs = [
        pl.BlockSpec((_G, _N, 128), lambda i: (i, 0, 0)),
        pl.BlockSpec((_G, _N, 128), lambda i: (i, 0, 0)),
        pl.BlockSpec((_G, _N, 128), lambda i: (i, 0, 0)),
        pl.BlockSpec((_G, 1), lambda i: (i, 0)),
    ]
    out_shapes = [
        jax.ShapeDtypeStruct((_B, _N, 128), jnp.float32),
        jax.ShapeDtypeStruct((_B, _N, 128), jnp.float32),
        jax.ShapeDtypeStruct((_B, _N, 128), jnp.float32),
        jax.ShapeDtypeStruct((_B, 1), jnp.float32),
    ]
    mu3, lv3, z3, pred = pl.pallas_call(
        _body,
        grid=grid,
        in_specs=in_specs,
        out_specs=out_specs,
        out_shape=out_shapes,
        compiler_params=pltpu.CompilerParams(
            dimension_semantics=("parallel",)),
    )(cell_subs, ds3, W_cell, b_cell, W_gnn, b_gnn.reshape(1, 128), weight,
      W1, b1.reshape(1, 256),
      jnp.concatenate([W_mu, W_lv], axis=1),
      jnp.concatenate([b_mu, b_lv]).reshape(1, 256),
      jnp.concatenate([Wq / jnp.sqrt(jnp.float32(_DH)), Wk, Wv], axis=1), Wo,
      Wp1, bp1.reshape(1, 256), Wp2, bp2.reshape(1, 1))

    return (pred, mu3.reshape(-1, 128), lv3.reshape(-1, 128),
            z3.reshape(-1, 128))


# speedup vs baseline: 1.2639x; 1.1714x over previous
import jax
import jax.numpy as jnp
from jax.experimental import pallas as pl
from jax.experimental.pallas import tpu as pltpu

_B = 256
_ND = 64
_NC = 8
_N = _ND + _NC
_G = 64
_HEADS = 4
_DH = 32


def _body(cs_ref, ds_ref, Wc_ref, bc_ref, Wg_ref, bg_ref, wgt_ref,
          W1_ref, b1_ref, Wmu_ref, bmu_ref, Wlv_ref, blv_ref,
          Wq_ref, Wk_ref, Wv_ref, Wo_ref, Wp1_ref, bp1_ref, Wp2_ref, bp2_ref,
          mu_ref, lv_ref, z_ref, pred_ref):
    G = _G
    relu = lambda x: jnp.maximum(x, 0.0)
    bf16 = jnp.bfloat16
    f32 = jnp.float32

    def bdot(a, b, dn=None):
        a16, b16 = a.astype(bf16), b.astype(bf16)
        if dn is None:
            dn = (((a.ndim - 1,), (0,)), ((), ()))
        return jax.lax.dot_general(a16, b16, dn, preferred_element_type=f32)

    def fdot(a, b, dn=None):
        if dn is None:
            dn = (((a.ndim - 1,), (0,)), ((), ()))
        return jax.lax.dot_general(a, b, dn, preferred_element_type=f32)

    cell_list = []
    for c in range(_NC):
        cell_list.append(relu(bdot(cs_ref[c], Wc_ref[c]) + bc_ref[c:c + 1, :]))
    cell3 = jnp.concatenate([x[:, None, :] for x in cell_list], axis=1)

    ds = ds_ref[...]
    sub = relu(bdot(ds.reshape(G * _ND, 128), Wg_ref[...]) + bg_ref[...])
    drug3 = sub.reshape(G, _ND, 128)

    T3 = jax.lax.dot_general(cell3.astype(bf16), wgt_ref[...].astype(bf16),
                             (((2,), (1,)), ((), ())),
                             preferred_element_type=f32)
    s3 = jax.lax.dot_general(T3.astype(bf16), drug3.astype(bf16),
                             (((2,), (2,)), ((0,), (0,))),
                             preferred_element_type=f32)
    gw = jax.nn.sigmoid(s3)
    wmT = jnp.where(gw >= 0.5, gw, 0.0)
    sig = wmT

    dinv_d = jax.lax.rsqrt(1.0 + jnp.sum(sig, axis=1))[:, :, None]
    dinv_c = jax.lax.rsqrt(1.0 + jnp.sum(sig, axis=2))[:, :, None]

    def propagate(Xd3, Xc3):
        Yd = dinv_d * Xd3
        Yc = dinv_c * Xc3
        AXd = dinv_d * (Yd + fdot(wmT, Yc, (((1,), (1,)), ((0,), (0,)))))
        AXc = dinv_c * (Yc + fdot(wmT, Yd, (((2,), (1,)), ((0,), (0,)))))
        return jnp.concatenate([AXd, AXc], axis=1)

    AX = propagate(drug3, cell3).reshape(G * _N, 128)
    H = relu(fdot(AX, W1_ref[...]) + b1_ref[...])
    H3 = H.reshape(G, _N, 256)

    A2 = propagate(H3[:, :_ND, :], H3[:, _ND:, :]).reshape(G * _N, 256)
    mu = fdot(A2, Wmu_ref[...]) + bmu_ref[...]
    lv = fdot(A2, Wlv_ref[...]) + blv_ref[...]
    mu_ref[...] = mu.reshape(G, _N, 128)
    z_ref[...] = mu.reshape(G, _N, 128)
    lv_ref[...] = lv.reshape(G, _N, 128)

    scale = 1.0 / jnp.sqrt(jnp.float32(_DH))
    qa = (fdot(mu, Wq_ref[...]) * scale).reshape(G, _N, 128)
    ka = fdot(mu, Wk_ref[...]).reshape(G, _N, 128)
    va = fdot(mu, Wv_ref[...]).reshape(G, _N, 128)
    parts = []
    for h in range(_HEADS):
        q3 = qa[:, :, h * _DH:(h + 1) * _DH]
        k3 = ka[:, :, h * _DH:(h + 1) * _DH]
        v3 = va[:, :, h * _DH:(h + 1) * _DH]
        S = fdot(q3, k3, (((2,), (2,)), ((0,), (0,))))
        S = S - jnp.max(S, axis=-1, keepdims=True)
        E = jnp.exp(S)
        rinv = (1.0 / _N) / jnp.sum(E, axis=-1)
        abar = fdot(rinv, E, (((1,), (1,)), ((0,), (0,))))
        parts.append(fdot(abar, v3, (((1,), (1,)), ((0,), (0,)))))
    obar = jnp.concatenate(parts, axis=-1)

    hg = fdot(obar, Wo_ref[...])
    p1 = relu(fdot(hg, Wp1_ref[...]) + bp1_ref[...])
    pred_ref[...] = jax.nn.sigmoid(fdot(p1, Wp2_ref[...]) + bp2_ref[...])


def kernel(cell_subs, drug_subs, batch, drug_cell_batch, W_cell, b_cell,
           W_gnn, b_gnn, weight, W1, b1, W_mu, b_mu, W_lv, b_lv,
           Wq, Wk, Wv, Wo, Wp1, bp1, Wp2, bp2):
    del batch, drug_cell_batch
    ds3 = drug_subs.reshape(_B, _ND, 128)
    grid = (_B // _G,)

    def full(a):
        return pl.BlockSpec(a.shape, lambda i: (0,) * a.ndim)

    in_specs = [
        pl.BlockSpec((_NC, _G, 128), lambda i: (0, i, 0)),
        pl.BlockSpec((_G, _ND, 128), lambda i: (i, 0, 0)),
        full(W_cell), full(b_cell),
        full(W_gnn), pl.BlockSpec((1, 128), lambda i: (0, 0)),
        full(weight),
        full(W1), pl.BlockSpec((1, 256), lambda i: (0, 0)),
        full(W_mu), pl.BlockSpec((1, 128), lambda i: (0, 0)),
        full(W_lv), pl.BlockSpec((1, 128), lambda i: (0, 0)),
        full(Wq), full(Wk), full(Wv), full(Wo),
        full(Wp1), pl.BlockSpec((1, 256), lambda i: (0, 0)),
        full(Wp2), pl.BlockSpec((1, 1), lambda i: (0, 0)),
    ]
    out_specs = [
        pl.BlockSpec((_G, _N, 128), lambda i: (i, 0, 0)),
        pl.BlockSpec((_G, _N, 128), lambda i: (i, 0, 0)),
        pl.BlockSpec((_G, _N, 128), lambda i: (i, 0, 0)),
        pl.BlockSpec((_G, 1), lambda i: (i, 0)),
    ]
    out_shapes = [
        jax.ShapeDtypeStruct((_B, _N, 128), jnp.float32),
        jax.ShapeDtypeStruct((_B, _N, 128), jnp.float32),
        jax.ShapeDtypeStruct((_B, _N, 128), jnp.float32),
        jax.ShapeDtypeStruct((_B, 1), jnp.float32),
    ]
    mu3, lv3, z3, pred = pl.pallas_call(
        _body,
        grid=grid,
        in_specs=in_specs,
        out_specs=out_specs,
        out_shape=out_shapes,
        compiler_params=pltpu.CompilerParams(
            dimension_semantics=("parallel",)),
    )(cell_subs, ds3, W_cell, b_cell, W_gnn, b_gnn.reshape(1, 128), weight,
      W1, b1.reshape(1, 256), W_mu, b_mu.reshape(1, 128),
      W_lv, b_lv.reshape(1, 128), Wq, Wk, Wv, Wo,
      Wp1, bp1.reshape(1, 256), Wp2, bp2.reshape(1, 1))

    return (pred, mu3.reshape(-1, 128), lv3.reshape(-1, 128),
            z3.reshape(-1, 128))


# speedup vs baseline: 1.2648x; 1.0007x over previous
import jax
import jax.numpy as jnp
from jax.experimental import pallas as pl
from jax.experimental.pallas import tpu as pltpu

_B = 256
_ND = 64
_NC = 8
_N = _ND + _NC
_G = 64
_HEADS = 4
_DH = 32


def _body(cs_ref, ds_ref, Wc_ref, bc_ref, Wg_ref, bg_ref, wgt_ref,
          W1_ref, b1_ref, Wmu_ref, bmu_ref, Wlv_ref, blv_ref,
          Wq_ref, Wk_ref, Wv_ref, Wo_ref, Wp1_ref, bp1_ref, Wp2_ref, bp2_ref,
          mu_ref, lv_ref, z_ref, pred_ref):
    G = _G
    relu = lambda x: jnp.maximum(x, 0.0)
    bf16 = jnp.bfloat16
    f32 = jnp.float32

    def bdot(a, b, dn=None):
        a16, b16 = a.astype(bf16), b.astype(bf16)
        if dn is None:
            dn = (((a.ndim - 1,), (0,)), ((), ()))
        return jax.lax.dot_general(a16, b16, dn, preferred_element_type=f32)

    def fdot(a, b, dn=None):
        if dn is None:
            dn = (((a.ndim - 1,), (0,)), ((), ()))
        return jax.lax.dot_general(a, b, dn, preferred_element_type=f32)

    cell_list = []
    for c in range(_NC):
        cell_list.append(relu(bdot(cs_ref[c], Wc_ref[c]) + bc_ref[c:c + 1, :]))
    cell3 = jnp.concatenate([x[:, None, :] for x in cell_list], axis=1)

    ds = ds_ref[...]
    sub = relu(bdot(ds.reshape(G * _ND, 128), Wg_ref[...]) + bg_ref[...])
    drug3 = sub.reshape(G, _ND, 128)

    T3 = jax.lax.dot_general(cell3.astype(bf16), wgt_ref[...].astype(bf16),
                             (((2,), (1,)), ((), ())),
                             preferred_element_type=f32)
    s3 = jax.lax.dot_general(T3.astype(bf16), drug3.astype(bf16),
                             (((2,), (2,)), ((0,), (0,))),
                             preferred_element_type=f32)
    gw = jax.nn.sigmoid(s3)
    wmT = jnp.where(gw >= 0.5, gw, 0.0)
    sig = wmT

    dinv_d = jax.lax.rsqrt(1.0 + jnp.sum(sig, axis=1))[:, :, None]
    dinv_c = jax.lax.rsqrt(1.0 + jnp.sum(sig, axis=2))[:, :, None]

    def propagate(Xd3, Xc3):
        Yd = dinv_d * Xd3
        Yc = dinv_c * Xc3
        AXd = dinv_d * (Yd + fdot(wmT, Yc, (((1,), (1,)), ((0,), (0,)))))
        AXc = dinv_c * (Yc + fdot(wmT, Yd, (((2,), (1,)), ((0,), (0,)))))
        return jnp.concatenate([AXd, AXc], axis=1)

    AX = propagate(drug3, cell3).reshape(G * _N, 128)
    H = relu(fdot(AX, W1_ref[...]) + b1_ref[...])
    H3 = H.reshape(G, _N, 256)

    A2 = propagate(H3[:, :_ND, :], H3[:, _ND:, :]).reshape(G * _N, 256)
    mu = fdot(A2, Wmu_ref[...]) + bmu_ref[...]
    lv = fdot(A2, Wlv_ref[...]) + blv_ref[...]
    mu_ref[...] = mu.reshape(G, _N, 128)
    z_ref[...] = mu.reshape(G, _N, 128)
    lv_ref[...] = lv.reshape(G, _N, 128)

    scale = 1.0 / jnp.sqrt(jnp.float32(_DH))
    qa = fdot(mu, Wq_ref[...] * scale).reshape(G, _N, 128)
    ka = fdot(mu, Wk_ref[...]).reshape(G, _N, 128)
    va = fdot(mu, Wv_ref[...]).reshape(G, _N, 128)
    parts = []
    for h in range(_HEADS):
        q3 = qa[:, :, h * _DH:(h + 1) * _DH]
        k3 = ka[:, :, h * _DH:(h + 1) * _DH]
        v3 = va[:, :, h * _DH:(h + 1) * _DH]
        S = fdot(q3, k3, (((2,), (2,)), ((0,), (0,))))
        S = S - jnp.max(S, axis=-1, keepdims=True)
        E = jnp.exp(S)
        rinv = (1.0 / _N) / jnp.sum(E, axis=-1)
        abar = fdot(rinv, E, (((1,), (1,)), ((0,), (0,))))
        parts.append(fdot(abar, v3, (((1,), (1,)), ((0,), (0,)))))
    obar = jnp.concatenate(parts, axis=-1)

    hg = fdot(obar, Wo_ref[...])
    p1 = relu(fdot(hg, Wp1_ref[...]) + bp1_ref[...])
    pred_ref[...] = jax.nn.sigmoid(fdot(p1, Wp2_ref[...]) + bp2_ref[...])


def kernel(cell_subs, drug_subs, batch, drug_cell_batch, W_cell, b_cell,
           W_gnn, b_gnn, weight, W1, b1, W_mu, b_mu, W_lv, b_lv,
           Wq, Wk, Wv, Wo, Wp1, bp1, Wp2, bp2):
    del batch, drug_cell_batch
    ds3 = drug_subs.reshape(_B, _ND, 128)
    grid = (_B // _G,)

    def full(a):
        return pl.BlockSpec(a.shape, lambda i: (0,) * a.ndim)

    in_specs = [
        pl.BlockSpec((_NC, _G, 128), lambda i: (0, i, 0)),
        pl.BlockSpec((_G, _ND, 128), lambda i: (i, 0, 0)),
        full(W_cell), full(b_cell),
        full(W_gnn), pl.BlockSpec((1, 128), lambda i: (0, 0)),
        full(weight),
        full(W1), pl.BlockSpec((1, 256), lambda i: (0, 0)),
        full(W_mu), pl.BlockSpec((1, 128), lambda i: (0, 0)),
        full(W_lv), pl.BlockSpec((1, 128), lambda i: (0, 0)),
        full(Wq), full(Wk), full(Wv), full(Wo),
        full(Wp1), pl.BlockSpec((1, 256), lambda i: (0, 0)),
        full(Wp2), pl.BlockSpec((1, 1), lambda i: (0, 0)),
    ]
    out_specs = [
        pl.BlockSpec((_G, _N, 128), lambda i: (i, 0, 0)),
        pl.BlockSpec((_G, _N, 128), lambda i: (i, 0, 0)),
        pl.BlockSpec((_G, _N, 128), lambda i: (i, 0, 0)),
        pl.BlockSpec((_G, 1), lambda i: (i, 0)),
    ]
    out_shapes = [
        jax.ShapeDtypeStruct((_B, _N, 128), jnp.float32),
        jax.ShapeDtypeStruct((_B, _N, 128), jnp.float32),
        jax.ShapeDtypeStruct((_B, _N, 128), jnp.float32),
        jax.ShapeDtypeStruct((_B, 1), jnp.float32),
    ]
    mu3, lv3, z3, pred = pl.pallas_call(
        _body,
        grid=grid,
        in_specs=in_specs,
        out_specs=out_specs,
        out_shape=out_shapes,
        compiler_params=pltpu.CompilerParams(
            dimension_semantics=("parallel",)),
    )(cell_subs, ds3, W_cell, b_cell, W_gnn, b_gnn.reshape(1, 128), weight,
      W1, b1.reshape(1, 256), W_mu, b_mu.reshape(1, 128),
      W_lv, b_lv.reshape(1, 128), Wq, Wk, Wv, Wo,
      Wp1, bp1.reshape(1, 256), Wp2, bp2.reshape(1, 1))

    return (pred, mu3.reshape(-1, 128), lv3.reshape(-1, 128),
            z3.reshape(-1, 128))
